# baseline (speedup 1.0000x reference)
import functools

import jax
import jax.numpy as jnp
import numpy as np
from jax import lax
from jax.experimental import pallas as pl
from jax.experimental.pallas import tpu as pltpu
from jax.experimental.pallas import tpu_sc as plsc

_B, _C, _H, _W = 4, 96, 384, 384
_PAD = 1
_LANES = 16

_CGROUPS = 8
_CPG = _C // _CGROUPS

_RS = 64
_HALO = 8
_STRIPS = tuple(
    (r0, max(0, r0 - _HALO), min(_H, r0 + _RS + _HALO) - max(0, r0 - _HALO))
    for r0 in range(0, _H, _RS)
)
_IN_ROWS = _RS + 2 * _HALO


def _sc_gather(x_hbm, m_hbm, dummy_hbm, dummy_m_hbm, out_hbm, m_v, in_v0,
               in_v1, out_v0, out_v1, sems):
    in_bufs = (in_v0, in_v1)
    out_bufs = (out_v0, out_v1)
    info = plsc.get_sparse_core_info()
    nc = info.num_cores
    wid = lax.axis_index("s") * nc + lax.axis_index("c")
    b = wid // _CGROUPS
    cbase = (wid % _CGROUPS) * _CPG

    out_cp = [None, None]
    for r0, st, nst in _STRIPS:

        def row0(j):
            return (b * _C + (cbase + j)) * _H

        def issue_rows(j, buf, sem_idx, st=st, nst=nst):
            base = row0(j) + st

            def body(k, carry):
                pltpu.async_copy(x_hbm.at[base + k],
                                 in_bufs[buf].at[pl.ds(k * _W, _W)],
                                 sems.at[sem_idx])
                return carry

            lax.fori_loop(0, nst, body, 0)

        issue_rows(0, 0, 0)
        m_base = b * (_H // 2) + r0 // 2

        def m_body(k, carry):
            pltpu.async_copy(m_hbm.at[m_base + k],
                             m_v.at[pl.ds(k * _W, _W)], sems.at[4])
            return carry

        lax.fori_loop(0, _RS // 2, m_body, 0)
        pltpu.make_async_copy(dummy_m_hbm, m_v, sems.at[4]).wait()

        for j in range(_CPG):
            cur = j & 1
            nxt = cur ^ 1
            if j + 1 < _CPG:
                issue_rows(j + 1, nxt, nxt)
            pltpu.make_async_copy(
                dummy_hbm.at[pl.ds(0, nst * _W)],
                in_bufs[cur].at[pl.ds(0, nst * _W)],
                sems.at[cur]).wait()
            if out_cp[cur] is not None:
                out_cp[cur].wait()

            @plsc.parallel_loop(0, (_RS // 2) * _W, step=_LANES, unroll=8)
            def gather_body(off, cur=cur):
                h = off // (_W // 2)
                rem = off - h * (_W // 2)
                v = m_v[pl.ds(off, _LANES)]
                lo = lax.bitwise_and(v, 0xFFFF)
                hi = lax.shift_right_logical(v, 16)
                out_bufs[cur][h, pl.ds(2 * rem, _LANES)] = (
                    plsc.load_gather(in_bufs[cur], [lo]))
                out_bufs[cur][h, pl.ds(2 * rem + _LANES, _LANES)] = (
                    plsc.load_gather(in_bufs[cur], [hi]))
            out_cp[cur] = pltpu.async_copy(
                out_bufs[cur],
                out_hbm.at[pl.ds(row0(j) + r0, _RS), :],
                sems.at[2 + cur])
    for cp in out_cp:
        if cp is not None:
            cp.wait()


_M_CACHE = [None]


def _index_map(B, H, W):
    if _M_CACHE[0] is not None:
        return _M_CACHE[0]
    with jax.ensure_compile_time_eval():
        kh = jax.random.fold_in(jax.random.key(0), 1)
        kw = jax.random.fold_in(jax.random.key(0), 2)
        dh = jax.random.randint(kh, (B, 1, H, W), -_PAD, _PAD + 1)
        dw = jax.random.randint(kw, (B, 1, H, W), -_PAD, _PAD + 1)
        row = jnp.arange(H)[None, None, :, None] + dh
        col = jnp.arange(W)[None, None, None, :] + dw
        row = (H - 1) - jnp.abs((H - 1) - jnp.abs(row))
        col = (W - 1) - jnp.abs((W - 1) - jnp.abs(col))
        st_h = jnp.maximum(0, (jnp.arange(H) // _RS) * _RS - _HALO)
        mloc = ((row - st_h[None, None, :, None]) * W + col).astype(jnp.int32)
    mloc = np.asarray(mloc).reshape(B * H, W)
    r = mloc.reshape(B * H, W // 32, 2, _LANES)
    packed = (r[:, :, 0, :] | (r[:, :, 1, :] << 16)).astype(np.int32)
    _M_CACHE[0] = packed.reshape(B * H // 2, W)
    return _M_CACHE[0]


@jax.jit
def kernel(x):
    B, C, H, W = x.shape
    m = jnp.asarray(_index_map(B, H, W))

    sc = functools.partial(
        pl.kernel,
        out_type=jax.ShapeDtypeStruct((B * C * H, W), jnp.float32),
        mesh=plsc.VectorSubcoreMesh(core_axis_name="c", subcore_axis_name="s"),
        compiler_params=pltpu.CompilerParams(needs_layout_passes=False),
        scratch_types=[
            pltpu.VMEM(((_RS // 2) * _W,), jnp.int32),
            pltpu.VMEM((_IN_ROWS * _W,), jnp.float32),
            pltpu.VMEM((_IN_ROWS * _W,), jnp.float32),
            pltpu.VMEM((_RS, _W), jnp.float32),
            pltpu.VMEM((_RS, _W), jnp.float32),
            pltpu.SemaphoreType.DMA((4,)),
        ],
    )(_sc_gather)
    dummy = jnp.zeros((_IN_ROWS * _W,), jnp.float32)
    dummy_m = jnp.zeros(((_RS // 2) * _W,), jnp.int32)
    out = sc(x.reshape(B * C * H, W), m, dummy, dummy_m)
    return out.reshape(B, C, H, W)


# speedup vs baseline: 1.0198x; 1.0198x over previous
import functools

import jax
import jax.numpy as jnp
import numpy as np
from jax import lax
from jax.experimental import pallas as pl
from jax.experimental.pallas import tpu as pltpu
from jax.experimental.pallas import tpu_sc as plsc

_B, _C, _H, _W = 4, 96, 384, 384
_PAD = 1
_LANES = 16

_CGROUPS = 8
_CPG = _C // _CGROUPS

_RS = 64
_HALO = 8
_STRIPS = tuple(
    (r0, max(0, r0 - _HALO), min(_H, r0 + _RS + _HALO) - max(0, r0 - _HALO))
    for r0 in range(0, _H, _RS)
)
_IN_ROWS = _RS + 2 * _HALO


def _sc_gather(x_hbm, m_hbm, dummy_hbm, dummy_m_hbm, out_hbm, m_v, in_v0,
               in_v1, out_v0, out_v1, sems):
    in_bufs = (in_v0, in_v1)
    out_bufs = (out_v0, out_v1)
    info = plsc.get_sparse_core_info()
    nc = info.num_cores
    wid = lax.axis_index("s") * nc + lax.axis_index("c")
    b = wid // _CGROUPS
    cbase = (wid % _CGROUPS) * _CPG

    out_cp = [None, None]
    for r0, st, nst in _STRIPS:

        def row0(j):
            return (b * _C + (cbase + j)) * _H

        def issue_rows(j, buf, sem_idx, st=st, nst=nst):
            base = row0(j) + st

            def body(k, carry):
                pltpu.async_copy(x_hbm.at[base + k],
                                 in_bufs[buf].at[pl.ds(k * _W, _W)],
                                 sems.at[sem_idx])
                return carry

            lax.fori_loop(0, nst, body, 0)

        issue_rows(0, 0, 0)
        m_base = b * (_H // 2) + r0 // 2

        def m_body(k, carry):
            pltpu.async_copy(m_hbm.at[m_base + k],
                             m_v.at[pl.ds(k * _W, _W)], sems.at[4])
            return carry

        lax.fori_loop(0, _RS // 2, m_body, 0)
        pltpu.make_async_copy(dummy_m_hbm, m_v, sems.at[4]).wait()

        for j in range(_CPG):
            cur = j & 1
            nxt = cur ^ 1
            if j + 1 < _CPG:
                issue_rows(j + 1, nxt, nxt)
            pltpu.make_async_copy(
                dummy_hbm.at[pl.ds(0, nst * _W)],
                in_bufs[cur].at[pl.ds(0, nst * _W)],
                sems.at[cur]).wait()
            if out_cp[cur] is not None:
                out_cp[cur].wait()

            @plsc.parallel_loop(0, (_RS // 2) * _W, step=_LANES, unroll=4)
            def gather_body(off, cur=cur):
                h = off // (_W // 2)
                rem = off - h * (_W // 2)
                v = m_v[pl.ds(off, _LANES)]
                lo = lax.bitwise_and(v, 0xFFFF)
                hi = lax.shift_right_logical(v, 16)
                out_bufs[cur][h, pl.ds(2 * rem, _LANES)] = (
                    plsc.load_gather(in_bufs[cur], [lo]))
                out_bufs[cur][h, pl.ds(2 * rem + _LANES, _LANES)] = (
                    plsc.load_gather(in_bufs[cur], [hi]))
            out_cp[cur] = pltpu.async_copy(
                out_bufs[cur],
                out_hbm.at[pl.ds(row0(j) + r0, _RS), :],
                sems.at[2 + cur])
    for cp in out_cp:
        if cp is not None:
            cp.wait()


_M_CACHE = [None]


def _index_map(B, H, W):
    if _M_CACHE[0] is not None:
        return _M_CACHE[0]
    with jax.ensure_compile_time_eval():
        kh = jax.random.fold_in(jax.random.key(0), 1)
        kw = jax.random.fold_in(jax.random.key(0), 2)
        dh = jax.random.randint(kh, (B, 1, H, W), -_PAD, _PAD + 1)
        dw = jax.random.randint(kw, (B, 1, H, W), -_PAD, _PAD + 1)
        row = jnp.arange(H)[None, None, :, None] + dh
        col = jnp.arange(W)[None, None, None, :] + dw
        row = (H - 1) - jnp.abs((H - 1) - jnp.abs(row))
        col = (W - 1) - jnp.abs((W - 1) - jnp.abs(col))
        st_h = jnp.maximum(0, (jnp.arange(H) // _RS) * _RS - _HALO)
        mloc = ((row - st_h[None, None, :, None]) * W + col).astype(jnp.int32)
    mloc = np.asarray(mloc).reshape(B * H, W)
    r = mloc.reshape(B * H, W // 32, 2, _LANES)
    packed = (r[:, :, 0, :] | (r[:, :, 1, :] << 16)).astype(np.int32)
    _M_CACHE[0] = packed.reshape(B * H // 2, W)
    return _M_CACHE[0]


@jax.jit
def kernel(x):
    B, C, H, W = x.shape
    m = jnp.asarray(_index_map(B, H, W))

    sc = functools.partial(
        pl.kernel,
        out_type=jax.ShapeDtypeStruct((B * C * H, W), jnp.float32),
        mesh=plsc.VectorSubcoreMesh(core_axis_name="c", subcore_axis_name="s"),
        compiler_params=pltpu.CompilerParams(needs_layout_passes=False),
        scratch_types=[
            pltpu.VMEM(((_RS // 2) * _W,), jnp.int32),
            pltpu.VMEM((_IN_ROWS * _W,), jnp.float32),
            pltpu.VMEM((_IN_ROWS * _W,), jnp.float32),
            pltpu.VMEM((_RS, _W), jnp.float32),
            pltpu.VMEM((_RS, _W), jnp.float32),
            pltpu.SemaphoreType.DMA((4,)),
        ],
    )(_sc_gather)
    dummy = jnp.zeros((_IN_ROWS * _W,), jnp.float32)
    dummy_m = jnp.zeros(((_RS // 2) * _W,), jnp.int32)
    out = sc(x.reshape(B * C * H, W), m, dummy, dummy_m)
    return out.reshape(B, C, H, W)
